# baseline (speedup 1.0000x reference)
import functools

import jax
import jax.numpy as jnp
from jax import lax
from jax.experimental import pallas as pl
from jax.experimental.pallas import tpu as pltpu
from jax.experimental.pallas import tpu_sc as plsc

N_NODES = 10000
N_EDGES = 320000
D = 128

NC = 2
NS = 16
NW = NC * NS

E_PER_TILE = N_EDGES // NW
EB = 125
CHUNKS = E_PER_TILE // EB
STAGE = 10
NSTAGE = CHUNKS // STAGE
PAIRS = STAGE // 2
DEG_INFLIGHT = 5
DC = 104
N_PAD = 10112
ROWS_PER_TILE = N_PAD // NS
_DRAIN_CHUNKS = [(k * DC, DC) for k in range(ROWS_PER_TILE // DC)]
if ROWS_PER_TILE % DC:
    _DRAIN_CHUNKS.append((ROWS_PER_TILE - ROWS_PER_TILE % DC, ROWS_PER_TILE % DC))

_sc_mesh = plsc.VectorSubcoreMesh(core_axis_name="c", subcore_axis_name="s")


@functools.partial(
    pl.kernel,
    mesh=_sc_mesh,
    out_type=[
        jax.ShapeDtypeStruct((NC, N_PAD, D), jnp.float32),
        jax.ShapeDtypeStruct((NC, N_PAD, 16), jnp.float32),
    ],
    scratch_types=[
        pltpu.VMEM((STAGE, EB), jnp.int32),
        pltpu.VMEM((STAGE, EB), jnp.int32),
        pltpu.VMEM((EB, D), jnp.float32),
        pltpu.VMEM((EB, D), jnp.float32),
        pltpu.VMEM((DC, 16), jnp.float32),
        pltpu.VMEM_SHARED((N_PAD, D), jnp.float32),
        pltpu.SemaphoreType.DMA,
        pltpu.SemaphoreType.DMA,
        pltpu.SemaphoreType.DMA,
    ],
)
def _sc_aggregate(x_hbm, row_hbm, col_hbm, zeros_hbm, ones_hbm,
                  agg_out, deg_out,
                  row_v, col_v, gbufa, gbufb, cbuf, agg_sh, sema, semb, sems):
    c = lax.axis_index("c")
    s = lax.axis_index("s")
    wid = c * NS + s
    r0 = s * ROWS_PER_TILE

    def zero_accum():
        pltpu.sync_copy(zeros_hbm, gbufa)
        for off, sz in _DRAIN_CHUNKS:
            pltpu.sync_copy(gbufa.at[pl.ds(0, sz)],
                            agg_sh.at[pl.ds(r0 + off, sz)])

    def drain_accum(out_ref):
        for off, sz in _DRAIN_CHUNKS:
            pltpu.sync_copy(agg_sh.at[pl.ds(r0 + off, sz)],
                            gbufa.at[pl.ds(0, sz)])
            pltpu.sync_copy(gbufa.at[pl.ds(0, sz)],
                            out_ref.at[c, pl.ds(r0 + off, sz)])

    zero_accum()
    plsc.subcore_barrier()

    def stage_body(g, carry):
        pltpu.sync_copy(row_hbm.at[wid, g], row_v)
        pltpu.sync_copy(col_hbm.at[wid, g], col_v)
        pltpu.async_copy(x_hbm.at[col_v.at[0]], gbufa, sema)

        def body(t, carry2):
            j0 = 2 * t
            j1 = j0 + 1
            pltpu.make_async_copy(x_hbm.at[col_v.at[j0]], gbufa, sema).wait()
            pltpu.async_copy(x_hbm.at[col_v.at[j1]], gbufb, semb)
            pltpu.sync_copy(gbufa, agg_sh.at[row_v.at[j0]], add=True)
            pltpu.make_async_copy(x_hbm.at[col_v.at[j1]], gbufb, semb).wait()

            @pl.when(t < PAIRS - 1)
            def _():
                pltpu.async_copy(x_hbm.at[col_v.at[j0 + 2]], gbufa, sema)

            pltpu.sync_copy(gbufb, agg_sh.at[row_v.at[j1]], add=True)
            return carry2

        lax.fori_loop(0, PAIRS, body, 0)
        return carry

    lax.fori_loop(0, NSTAGE, stage_body, 0)
    plsc.subcore_barrier()

    pltpu.sync_copy(zeros_hbm, gbufb)
    for off, sz in _DRAIN_CHUNKS:
        pltpu.sync_copy(agg_sh.at[pl.ds(r0 + off, sz)], gbufa.at[pl.ds(0, sz)])
        pltpu.sync_copy(gbufa.at[pl.ds(0, sz)],
                        agg_out.at[c, pl.ds(r0 + off, sz)])
        pltpu.sync_copy(gbufb.at[pl.ds(0, sz)],
                        agg_sh.at[pl.ds(r0 + off, sz)])

    pltpu.sync_copy(ones_hbm, gbufb)
    plsc.subcore_barrier()

    def stage_body_deg(g, carry):
        pltpu.sync_copy(row_hbm.at[wid, g], row_v)
        for p in range(DEG_INFLIGHT):
            pltpu.async_copy(gbufb, agg_sh.at[row_v.at[p]], sems, add=True)

        def body(j, carry2):
            pltpu.async_copy(
                gbufb, agg_sh.at[row_v.at[j + DEG_INFLIGHT]], sems, add=True)
            pltpu.make_async_copy(gbufb, agg_sh.at[row_v.at[j]], sems).wait()
            return carry2

        lax.fori_loop(0, STAGE - DEG_INFLIGHT, body, 0)
        for p in range(DEG_INFLIGHT):
            pltpu.make_async_copy(
                gbufb, agg_sh.at[row_v.at[STAGE - DEG_INFLIGHT + p]],
                sems).wait()
        return carry

    lax.fori_loop(0, NSTAGE, stage_body_deg, 0)
    plsc.subcore_barrier()
    for off, sz in _DRAIN_CHUNKS:
        pltpu.sync_copy(agg_sh.at[pl.ds(r0 + off, sz)],
                        gbufa.at[pl.ds(0, sz)])

        def compact(i, carry2):
            cbuf[i, pl.ds(0, 16)] = gbufa[i, pl.ds(0, 16)]
            return carry2

        lax.fori_loop(0, sz, compact, 0)
        pltpu.sync_copy(cbuf.at[pl.ds(0, sz)],
                        deg_out.at[c, pl.ds(r0 + off, sz)])


BLK = 1000


def _tc_body(agg_ref, deg_ref, w0_ref, b0_ref, w1_ref, b1_ref, y_ref):
    agg = agg_ref[0] + agg_ref[1]
    deg = deg_ref[0, :, 0:1] + deg_ref[1, :, 0:1]
    inv = 1.0 / jnp.maximum(deg, 1.0)
    mask = (deg > 0.0).astype(jnp.float32)
    h = jnp.dot(agg * inv, w0_ref[...], preferred_element_type=jnp.float32)
    h = jnp.maximum(h + b0_ref[...] * mask, 0.0)
    y = jnp.dot(h, w1_ref[...], preferred_element_type=jnp.float32)
    y_ref[...] = y + b1_ref[...]


def _tc_finish(agg, deg, w0, b0, w1, b1):
    grid = N_NODES // BLK
    return pl.pallas_call(
        _tc_body,
        grid=(grid,),
        in_specs=[
            pl.BlockSpec((NC, BLK, D), lambda i: (0, i, 0)),
            pl.BlockSpec((NC, BLK, 16), lambda i: (0, i, 0)),
            pl.BlockSpec((D, D), lambda i: (0, 0)),
            pl.BlockSpec((1, D), lambda i: (0, 0)),
            pl.BlockSpec((D, D), lambda i: (0, 0)),
            pl.BlockSpec((1, D), lambda i: (0, 0)),
        ],
        out_specs=pl.BlockSpec((BLK, D), lambda i: (i, 0)),
        out_shape=jax.ShapeDtypeStruct((N_NODES, D), jnp.float32),
    )(agg, deg, w0, b0, w1, b1)


def kernel(x, edge_index, W0, b0, W1, b1):
    ei = edge_index.astype(jnp.int32)
    row = ei[0].reshape(NW, NSTAGE, STAGE, EB)
    col = ei[1].reshape(NW, NSTAGE, STAGE, EB)
    zeros = jnp.zeros((EB, D), jnp.float32)
    ones = jnp.ones((EB, D), jnp.float32)
    agg, deg = _sc_aggregate(x, row, col, zeros, ones)
    return _tc_finish(agg, deg, W0, b0.reshape(1, D), W1, b1.reshape(1, D))


# speedup vs baseline: 1.0458x; 1.0458x over previous
import functools

import jax
import jax.numpy as jnp
from jax import lax
from jax.experimental import pallas as pl
from jax.experimental.pallas import tpu as pltpu
from jax.experimental.pallas import tpu_sc as plsc

N_NODES = 10000
N_EDGES = 320000
D = 128

NC = 2
NS = 16
NW = NC * NS

E_PER_TILE = N_EDGES // NW
EB = 125
CHUNKS = E_PER_TILE // EB
STAGE = 20
NSTAGE = CHUNKS // STAGE
PAIRS = STAGE // 2
DEG_INFLIGHT = 5
DC = 104
N_PAD = 10112
ROWS_PER_TILE = N_PAD // NS
_DRAIN_CHUNKS = [(k * DC, DC) for k in range(ROWS_PER_TILE // DC)]
if ROWS_PER_TILE % DC:
    _DRAIN_CHUNKS.append((ROWS_PER_TILE - ROWS_PER_TILE % DC, ROWS_PER_TILE % DC))

_sc_mesh = plsc.VectorSubcoreMesh(core_axis_name="c", subcore_axis_name="s")


@functools.partial(
    pl.kernel,
    mesh=_sc_mesh,
    out_type=[
        jax.ShapeDtypeStruct((NC, N_PAD, D), jnp.float32),
        jax.ShapeDtypeStruct((NC, N_PAD, D), jnp.float32),
    ],
    scratch_types=[
        pltpu.VMEM((STAGE, EB), jnp.int32),
        pltpu.VMEM((STAGE, EB), jnp.int32),
        pltpu.VMEM((EB, D), jnp.float32),
        pltpu.VMEM((EB, D), jnp.float32),
        pltpu.VMEM_SHARED((N_PAD, D), jnp.float32),
        pltpu.SemaphoreType.DMA,
        pltpu.SemaphoreType.DMA,
        pltpu.SemaphoreType.DMA,
    ],
)
def _sc_aggregate(x_hbm, row_hbm, col_hbm, zeros_hbm, ones_hbm,
                  agg_out, deg_out,
                  row_v, col_v, gbufa, gbufb, agg_sh, sema, semb, sems):
    c = lax.axis_index("c")
    s = lax.axis_index("s")
    wid = c * NS + s
    r0 = s * ROWS_PER_TILE

    def zero_accum():
        pltpu.sync_copy(zeros_hbm, gbufa)
        for off, sz in _DRAIN_CHUNKS:
            pltpu.sync_copy(gbufa.at[pl.ds(0, sz)],
                            agg_sh.at[pl.ds(r0 + off, sz)])

    def drain_accum(out_ref):
        for off, sz in _DRAIN_CHUNKS:
            pltpu.sync_copy(agg_sh.at[pl.ds(r0 + off, sz)],
                            gbufa.at[pl.ds(0, sz)])
            pltpu.sync_copy(gbufa.at[pl.ds(0, sz)],
                            out_ref.at[c, pl.ds(r0 + off, sz)])

    zero_accum()
    plsc.subcore_barrier()

    def stage_body(g, carry):
        pltpu.sync_copy(row_hbm.at[wid, g], row_v)
        pltpu.sync_copy(col_hbm.at[wid, g], col_v)
        pltpu.async_copy(x_hbm.at[col_v.at[0]], gbufa, sema)

        def body(t, carry2):
            j0 = 2 * t
            j1 = j0 + 1
            pltpu.make_async_copy(x_hbm.at[col_v.at[j0]], gbufa, sema).wait()
            pltpu.async_copy(x_hbm.at[col_v.at[j1]], gbufb, semb)
            pltpu.sync_copy(gbufa, agg_sh.at[row_v.at[j0]], add=True)
            pltpu.make_async_copy(x_hbm.at[col_v.at[j1]], gbufb, semb).wait()

            @pl.when(t < PAIRS - 1)
            def _():
                pltpu.async_copy(x_hbm.at[col_v.at[j0 + 2]], gbufa, sema)

            pltpu.sync_copy(gbufb, agg_sh.at[row_v.at[j1]], add=True)
            return carry2

        lax.fori_loop(0, PAIRS, body, 0)
        return carry

    lax.fori_loop(0, NSTAGE, stage_body, 0)
    plsc.subcore_barrier()

    pltpu.sync_copy(zeros_hbm, gbufb)
    for off, sz in _DRAIN_CHUNKS:
        pltpu.sync_copy(agg_sh.at[pl.ds(r0 + off, sz)], gbufa.at[pl.ds(0, sz)])
        pltpu.sync_copy(gbufa.at[pl.ds(0, sz)],
                        agg_out.at[c, pl.ds(r0 + off, sz)])
        pltpu.sync_copy(gbufb.at[pl.ds(0, sz)],
                        agg_sh.at[pl.ds(r0 + off, sz)])

    pltpu.sync_copy(ones_hbm, gbufb)
    plsc.subcore_barrier()

    def stage_body_deg(g, carry):
        pltpu.sync_copy(row_hbm.at[wid, g], row_v)
        for p in range(DEG_INFLIGHT):
            pltpu.async_copy(gbufb, agg_sh.at[row_v.at[p]], sems, add=True)

        def body(j, carry2):
            pltpu.async_copy(
                gbufb, agg_sh.at[row_v.at[j + DEG_INFLIGHT]], sems, add=True)
            pltpu.make_async_copy(gbufb, agg_sh.at[row_v.at[j]], sems).wait()
            return carry2

        lax.fori_loop(0, STAGE - DEG_INFLIGHT, body, 0)
        for p in range(DEG_INFLIGHT):
            pltpu.make_async_copy(
                gbufb, agg_sh.at[row_v.at[STAGE - DEG_INFLIGHT + p]],
                sems).wait()
        return carry

    lax.fori_loop(0, NSTAGE, stage_body_deg, 0)
    plsc.subcore_barrier()
    drain_accum(deg_out)


BLK = 1000


def _tc_body(agg_ref, deg_ref, w0_ref, b0_ref, w1_ref, b1_ref, y_ref):
    agg = agg_ref[0] + agg_ref[1]
    deg = deg_ref[0, :, 0:1] + deg_ref[1, :, 0:1]
    inv = 1.0 / jnp.maximum(deg, 1.0)
    mask = (deg > 0.0).astype(jnp.float32)
    h = jnp.dot(agg * inv, w0_ref[...], preferred_element_type=jnp.float32)
    h = jnp.maximum(h + b0_ref[...] * mask, 0.0)
    y = jnp.dot(h, w1_ref[...], preferred_element_type=jnp.float32)
    y_ref[...] = y + b1_ref[...]


def _tc_finish(agg, deg, w0, b0, w1, b1):
    grid = N_NODES // BLK
    return pl.pallas_call(
        _tc_body,
        grid=(grid,),
        in_specs=[
            pl.BlockSpec((NC, BLK, D), lambda i: (0, i, 0)),
            pl.BlockSpec((NC, BLK, D), lambda i: (0, i, 0)),
            pl.BlockSpec((D, D), lambda i: (0, 0)),
            pl.BlockSpec((1, D), lambda i: (0, 0)),
            pl.BlockSpec((D, D), lambda i: (0, 0)),
            pl.BlockSpec((1, D), lambda i: (0, 0)),
        ],
        out_specs=pl.BlockSpec((BLK, D), lambda i: (i, 0)),
        out_shape=jax.ShapeDtypeStruct((N_NODES, D), jnp.float32),
    )(agg, deg, w0, b0, w1, b1)


def kernel(x, edge_index, W0, b0, W1, b1):
    ei = edge_index.astype(jnp.int32)
    row = ei[0].reshape(NW, NSTAGE, STAGE, EB)
    col = ei[1].reshape(NW, NSTAGE, STAGE, EB)
    zeros = jnp.zeros((EB, D), jnp.float32)
    ones = jnp.ones((EB, D), jnp.float32)
    agg, deg = _sc_aggregate(x, row, col, zeros, ones)
    return _tc_finish(agg, deg, W0, b0.reshape(1, D), W1, b1.reshape(1, D))
